# baseline (speedup 1.0000x reference)
import functools

import jax
import jax.numpy as jnp
from jax import lax
from jax.experimental import pallas as pl
from jax.experimental.pallas import tpu as pltpu
from jax.experimental.pallas import tpu_sc as plsc

_NC = 2
_NS = 16
_NW = _NC * _NS

_CH = 128
_GRP = 6400
_VP = 102400


def _transform_pack(tableT, W, b):
    D, V = tableT.shape
    grid = _VP // (2 * _GRP)

    def body(tl_ref, tr_ref, w_ref, b_ref, o_ref):
        bias = b_ref[...][None, :]
        for h, t_ref in enumerate((tl_ref, tr_ref)):
            o_ref[:, h * D:(h + 1) * D] = (
                lax.dot_general(
                    t_ref[...],
                    w_ref[...],
                    dimension_numbers=(((0,), (0,)), ((), ())),
                    preferred_element_type=jnp.float32,
                )
                + bias
            )

    return pl.pallas_call(
        body,
        grid=(grid,),
        in_specs=[
            pl.BlockSpec((D, _GRP), lambda i: (0, 2 * i)),
            pl.BlockSpec((D, _GRP), lambda i: (0, 2 * i + 1)),
            pl.BlockSpec((D, D), lambda i: (0, 0)),
            pl.BlockSpec((D,), lambda i: (0,)),
        ],
        out_specs=pl.BlockSpec((_GRP, 2 * D), lambda i: (i, 0)),
        out_shape=jax.ShapeDtypeStruct((_VP // 2, 2 * D), jnp.float32),
    )(tableT, tableT, W, b)


def _gather(N, D):
    per = N // _NW
    NCH = per // _CH
    assert per % _CH == 0 and NCH % 2 == 0
    mesh = plsc.VectorSubcoreMesh(core_axis_name="c", subcore_axis_name="s")

    @functools.partial(
        pl.kernel,
        out_type=jax.ShapeDtypeStruct((N // _CH, _CH, D), jnp.float32),
        mesh=mesh,
        scratch_types=[
            pltpu.VMEM((NCH, _CH), jnp.int32),
            pltpu.VMEM((2, _CH, D), jnp.float32),
            pltpu.SemaphoreType.DMA,
            pltpu.SemaphoreType.DMA,
        ],
        compiler_params=pltpu.CompilerParams(use_tc_tiling_on_sc=False),
    )
    def gather_k(tok_hbm, t2_hbm, out_hbm, idx_v, rows_v, gsem, ssem):
        wid = lax.axis_index("s") * _NC + lax.axis_index("c")
        pltpu.sync_copy(tok_hbm.at[pl.ds(wid * NCH, NCH)], idx_v)

        def fire_g(c, p):
            pltpu.async_copy(t2_hbm.at[idx_v.at[c]], rows_v.at[p], gsem)

        def drain_g(c, p):
            pltpu.make_async_copy(
                t2_hbm.at[idx_v.at[c]], rows_v.at[p], gsem
            ).wait()

        def fire_s(c, p):
            pltpu.async_copy(rows_v.at[p], out_hbm.at[wid * NCH + c], ssem)

        def drain_s(c, p):
            pltpu.make_async_copy(
                rows_v.at[p], out_hbm.at[wid * NCH + c], ssem
            ).wait()

        fire_g(0, 0)
        drain_g(0, 0)
        fire_s(0, 0)
        fire_g(1, 1)

        def body(k, carry):
            c1 = 2 * k + 1
            drain_g(c1, 1)
            fire_s(c1, 1)
            drain_s(c1 - 1, 0)
            fire_g(c1 + 1, 0)
            c2 = 2 * k + 2
            drain_g(c2, 0)
            fire_s(c2, 0)
            drain_s(c2 - 1, 1)
            fire_g(c2 + 1, 1)
            return carry

        lax.fori_loop(0, NCH // 2 - 1, body, 0)

        drain_g(NCH - 1, 1)
        fire_s(NCH - 1, 1)
        drain_s(NCH - 2, 0)
        drain_s(NCH - 1, 1)

    return gather_k


def _untranspose(G128, L, D, Bsz):
    BH = Bsz // 2

    def body(g_ref, i_ref, o_ref):
        g = g_ref[...]
        for h in range(2):
            o_ref[0, :, h * BH:(h + 1) * BH] = lax.dot_general(
                i_ref[...],
                g[:, h * D:(h + 1) * D],
                dimension_numbers=(((0,), (1,)), ((), ())),
                preferred_element_type=jnp.float32,
            )

    return pl.pallas_call(
        body,
        grid=(L,),
        in_specs=[
            pl.BlockSpec((BH, 2 * D), lambda l: (l, 0)),
            pl.BlockSpec((D, D), lambda l: (0, 0)),
        ],
        out_specs=pl.BlockSpec((1, D, Bsz), lambda l: (l, 0, 0)),
        out_shape=jax.ShapeDtypeStruct((L, D, Bsz), jnp.float32),
    )(G128, jnp.eye(D, dtype=jnp.float32))


def kernel(token_ids, table, W, b):
    Bsz, L = token_ids.shape
    V, D = table.shape
    N = Bsz * L
    BH = Bsz // 2
    assert N % (_NW * _CH) == 0 and V <= _VP

    tokT = token_ids.T
    tokP = tokT.reshape(L, 2, BH).transpose(0, 2, 1).reshape(L, Bsz)
    k = tokP // (2 * _GRP)
    pos = tokP % (2 * _GRP)
    h = (pos >= _GRP).astype(jnp.int32)
    tokR = 2 * (k * _GRP + pos - _GRP * h) + h
    tok2d = tokR.reshape(N // _CH, _CH)

    t2p = _transform_pack(table.T, W, b)
    t2 = t2p.reshape(_VP, D)
    G3 = _gather(N, D)(tok2d, t2)
    G128 = G3.reshape(N // 2, 2 * D)
    outP = _untranspose(G128, L, D, Bsz)
    return jnp.transpose(outP, (2, 0, 1))


# speedup vs baseline: 1.2554x; 1.2554x over previous
import functools

import jax
import jax.numpy as jnp
from jax import lax
from jax.experimental import pallas as pl
from jax.experimental.pallas import tpu as pltpu
from jax.experimental.pallas import tpu_sc as plsc

_NC = 2
_NS = 16
_NW = _NC * _NS

_CH = 128
_GRP = 6400
_VP = 102400


def _transform_pack(tableT, W, b):
    D, V = tableT.shape
    grid = _VP // (2 * _GRP)

    def body(tl_ref, tr_ref, w_ref, b_ref, o_ref):
        bias = b_ref[...][None, :]
        for h, t_ref in enumerate((tl_ref, tr_ref)):
            o_ref[:, h * D:(h + 1) * D] = (
                lax.dot_general(
                    t_ref[...],
                    w_ref[...],
                    dimension_numbers=(((0,), (0,)), ((), ())),
                    preferred_element_type=jnp.float32,
                )
                + bias
            )

    return pl.pallas_call(
        body,
        grid=(grid,),
        in_specs=[
            pl.BlockSpec((D, _GRP), lambda i: (0, 2 * i)),
            pl.BlockSpec((D, _GRP), lambda i: (0, 2 * i + 1)),
            pl.BlockSpec((D, D), lambda i: (0, 0)),
            pl.BlockSpec((D,), lambda i: (0,)),
        ],
        out_specs=pl.BlockSpec((_GRP, 2 * D), lambda i: (i, 0)),
        out_shape=jax.ShapeDtypeStruct((_VP // 2, 2 * D), jnp.float32),
    )(tableT, tableT, W, b)


def _gather(N, D):
    per = N // _NW
    NCH = per // _CH
    assert per % _CH == 0 and NCH % 2 == 0
    mesh = plsc.VectorSubcoreMesh(core_axis_name="c", subcore_axis_name="s")

    @functools.partial(
        pl.kernel,
        out_type=jax.ShapeDtypeStruct((N // _CH, _CH, D), jnp.float32),
        mesh=mesh,
        scratch_types=[
            pltpu.VMEM((NCH, _CH), jnp.int32),
            pltpu.VMEM((2, _CH, D), jnp.float32),
            pltpu.SemaphoreType.DMA,
            pltpu.SemaphoreType.DMA,
        ],
        compiler_params=pltpu.CompilerParams(use_tc_tiling_on_sc=False),
    )
    def gather_k(tok_hbm, t2_hbm, out_hbm, idx_v, rows_v, gsem, ssem):
        wid = lax.axis_index("s") * _NC + lax.axis_index("c")
        pltpu.sync_copy(tok_hbm.at[pl.ds(wid * NCH, NCH)], idx_v)

        def fire_g(c, p):
            pltpu.async_copy(t2_hbm.at[idx_v.at[c]], rows_v.at[p], gsem)

        def drain_g(c, p):
            pltpu.make_async_copy(
                t2_hbm.at[idx_v.at[c]], rows_v.at[p], gsem
            ).wait()

        def fire_s(c, p):
            pltpu.async_copy(rows_v.at[p], out_hbm.at[wid * NCH + c], ssem)

        def drain_s(c, p):
            pltpu.make_async_copy(
                rows_v.at[p], out_hbm.at[wid * NCH + c], ssem
            ).wait()

        fire_g(0, 0)
        drain_g(0, 0)
        fire_s(0, 0)
        fire_g(1, 1)

        def body(k, carry):
            c1 = 2 * k + 1
            drain_g(c1, 1)
            fire_s(c1, 1)
            drain_s(c1 - 1, 0)
            fire_g(c1 + 1, 0)
            c2 = 2 * k + 2
            drain_g(c2, 0)
            fire_s(c2, 0)
            drain_s(c2 - 1, 1)
            fire_g(c2 + 1, 1)
            return carry

        lax.fori_loop(0, NCH // 2 - 1, body, 0)

        drain_g(NCH - 1, 1)
        fire_s(NCH - 1, 1)
        drain_s(NCH - 2, 0)
        drain_s(NCH - 1, 1)

    return gather_k


def _untranspose(G128, L, D, Bsz):
    BH = Bsz // 2

    def body(g_ref, i_ref, o_ref):
        g = g_ref[...]
        for h in range(2):
            o_ref[0, :, h * BH:(h + 1) * BH] = lax.dot_general(
                i_ref[...],
                g[:, h * D:(h + 1) * D],
                dimension_numbers=(((0,), (1,)), ((), ())),
                preferred_element_type=jnp.float32,
            )

    return pl.pallas_call(
        body,
        grid=(L,),
        in_specs=[
            pl.BlockSpec((BH, 2 * D), lambda l: (l, 0)),
            pl.BlockSpec((D, D), lambda l: (0, 0)),
        ],
        out_specs=pl.BlockSpec((1, D, Bsz), lambda l: (l, 0, 0)),
        out_shape=jax.ShapeDtypeStruct((L, D, Bsz), jnp.float32),
    )(G128, jnp.eye(D, dtype=jnp.float32))


def kernel(token_ids, table, W, b):
    Bsz, L = token_ids.shape
    V, D = table.shape
    N = Bsz * L
    BH = Bsz // 2
    assert N % (_NW * _CH) == 0 and V <= _VP

    tokT = token_ids.T
    k = tokT // (2 * _GRP)
    pos = tokT % (2 * _GRP)
    h = (pos >= _GRP).astype(jnp.int32)
    tokR = 2 * (k * _GRP + pos - _GRP * h) + h
    lo = jnp.repeat(tokR[:, :BH], 2, axis=1)
    hi = jnp.repeat(tokR[:, BH:], 2, axis=1)
    par = (jnp.arange(Bsz, dtype=jnp.int32) & 1).astype(bool)
    tokP = jnp.where(par[None, :], hi, lo)
    tok2d = tokP.reshape(N // _CH, _CH)

    t2p = _transform_pack(table.T, W, b)
    t2 = t2p.reshape(_VP, D)
    G3 = _gather(N, D)(tok2d, t2)
    G128 = G3.reshape(N // 2, 2 * D)
    outP = _untranspose(G128, L, D, Bsz)
    return jnp.transpose(outP, (2, 0, 1))


# speedup vs baseline: 2.0308x; 1.6176x over previous
import functools

import jax
import jax.numpy as jnp
from jax import lax
from jax.experimental import pallas as pl
from jax.experimental.pallas import tpu as pltpu
from jax.experimental.pallas import tpu_sc as plsc

_NC = 2
_NS = 16
_NW = _NC * _NS

_CH = 128
_GRP = 6400
_VP = 102400


def _transform_pack(tableT, W, b):
    D, V = tableT.shape
    grid = _VP // (2 * _GRP)

    def body(tl_ref, tr_ref, w_ref, b_ref, o_ref):
        bias = b_ref[...][None, :]
        for h, t_ref in enumerate((tl_ref, tr_ref)):
            o_ref[:, h * D:(h + 1) * D] = (
                lax.dot_general(
                    t_ref[...],
                    w_ref[...],
                    dimension_numbers=(((0,), (0,)), ((), ())),
                    preferred_element_type=jnp.float32,
                )
                + bias
            )

    return pl.pallas_call(
        body,
        grid=(grid,),
        in_specs=[
            pl.BlockSpec((D, _GRP), lambda i: (0, 2 * i)),
            pl.BlockSpec((D, _GRP), lambda i: (0, 2 * i + 1)),
            pl.BlockSpec((D, D), lambda i: (0, 0)),
            pl.BlockSpec((D,), lambda i: (0,)),
        ],
        out_specs=pl.BlockSpec((_GRP, 2 * D), lambda i: (i, 0)),
        out_shape=jax.ShapeDtypeStruct((_VP // 2, 2 * D), jnp.float32),
    )(tableT, tableT, W, b)


def _gather(N, D):
    per = N // _NW
    NCH = per // _CH
    assert per % _CH == 0 and NCH % 2 == 0
    mesh = plsc.VectorSubcoreMesh(core_axis_name="c", subcore_axis_name="s")

    HC = _CH // 2

    @functools.partial(
        pl.kernel,
        out_type=jax.ShapeDtypeStruct((N // _CH, HC, 2, D), jnp.float32),
        mesh=mesh,
        scratch_types=[
            pltpu.VMEM((NCH, _CH), jnp.int32),
            pltpu.VMEM((2, _CH, D), jnp.float32),
            pltpu.SemaphoreType.DMA,
            pltpu.SemaphoreType.DMA,
        ],
        compiler_params=pltpu.CompilerParams(use_tc_tiling_on_sc=False),
    )
    def gather_k(tok_hbm, t2_hbm, out_hbm, idx_v, rows_v, gsem, ssem):
        wid = lax.axis_index("s") * _NC + lax.axis_index("c")
        pltpu.sync_copy(tok_hbm.at[pl.ds(wid * NCH, NCH)], idx_v)

        def fire_g(c, p):
            pltpu.async_copy(t2_hbm.at[idx_v.at[c]], rows_v.at[p], gsem)

        def drain_g(c, p):
            pltpu.make_async_copy(
                t2_hbm.at[idx_v.at[c]], rows_v.at[p], gsem
            ).wait()

        def fire_s(c, p):
            for h in range(2):
                pltpu.async_copy(
                    rows_v.at[p, pl.ds(h * HC, HC)],
                    out_hbm.at[wid * NCH + c, :, h],
                    ssem,
                )

        def drain_s(c, p):
            for h in range(2):
                pltpu.make_async_copy(
                    rows_v.at[p, pl.ds(h * HC, HC)],
                    out_hbm.at[wid * NCH + c, :, h],
                    ssem,
                ).wait()

        fire_g(0, 0)
        drain_g(0, 0)
        fire_s(0, 0)
        fire_g(1, 1)

        def body(k, carry):
            c1 = 2 * k + 1
            drain_g(c1, 1)
            fire_s(c1, 1)
            drain_s(c1 - 1, 0)
            fire_g(c1 + 1, 0)
            c2 = 2 * k + 2
            drain_g(c2, 0)
            fire_s(c2, 0)
            drain_s(c2 - 1, 1)
            fire_g(c2 + 1, 1)
            return carry

        lax.fori_loop(0, NCH // 2 - 1, body, 0)

        drain_g(NCH - 1, 1)
        fire_s(NCH - 1, 1)
        drain_s(NCH - 2, 0)
        drain_s(NCH - 1, 1)

    return gather_k


def _untranspose(G128, L, D, Bsz):
    BH = Bsz // 2

    def body(g_ref, i_ref, o_ref):
        g = g_ref[...]
        for h in range(2):
            o_ref[0, :, h * BH:(h + 1) * BH] = lax.dot_general(
                i_ref[...],
                g[:, h * D:(h + 1) * D],
                dimension_numbers=(((0,), (1,)), ((), ())),
                preferred_element_type=jnp.float32,
            )

    return pl.pallas_call(
        body,
        grid=(L,),
        in_specs=[
            pl.BlockSpec((BH, 2 * D), lambda l: (l, 0)),
            pl.BlockSpec((D, D), lambda l: (0, 0)),
        ],
        out_specs=pl.BlockSpec((1, D, Bsz), lambda l: (l, 0, 0)),
        out_shape=jax.ShapeDtypeStruct((L, D, Bsz), jnp.float32),
    )(G128, jnp.eye(D, dtype=jnp.float32))


def kernel(token_ids, table, W, b):
    Bsz, L = token_ids.shape
    V, D = table.shape
    N = Bsz * L
    BH = Bsz // 2
    assert N % (_NW * _CH) == 0 and V <= _VP

    tokT = token_ids.T
    k = tokT // (2 * _GRP)
    pos = tokT % (2 * _GRP)
    h = (pos >= _GRP).astype(jnp.int32)
    tokR = 2 * (k * _GRP + pos - _GRP * h) + h
    HC = _CH // 2
    tok2d = (
        tokR.reshape(L, 2, BH // HC, HC)
        .transpose(0, 2, 1, 3)
        .reshape(N // _CH, _CH)
    )

    t2p = _transform_pack(table.T, W, b)
    t2 = t2p.reshape(_VP, D)
    G3 = _gather(N, D)(tok2d, t2)
    G128 = G3.reshape(N // 2, 2 * D)
    outP = _untranspose(G128, L, D, Bsz)
    return jnp.transpose(outP, (2, 0, 1))


# speedup vs baseline: 2.2316x; 1.0989x over previous
import functools

import jax
import jax.numpy as jnp
from jax import lax
from jax.experimental import pallas as pl
from jax.experimental.pallas import tpu as pltpu
from jax.experimental.pallas import tpu_sc as plsc

_NC = 2
_NS = 16
_NW = _NC * _NS

_CH = 128
_GRP = 6400
_VP = 102400


def _transform_pack(tableT, W, b):
    D, V = tableT.shape
    grid = _VP // (2 * _GRP)

    def body(tl_ref, tr_ref, w_ref, b_ref, o_ref):
        bias = b_ref[...][None, :]
        for h, t_ref in enumerate((tl_ref, tr_ref)):
            o_ref[:, h * D:(h + 1) * D] = (
                lax.dot_general(
                    t_ref[...],
                    w_ref[...],
                    dimension_numbers=(((0,), (0,)), ((), ())),
                    preferred_element_type=jnp.float32,
                )
                + bias
            )

    return pl.pallas_call(
        body,
        grid=(grid,),
        in_specs=[
            pl.BlockSpec((D, _GRP), lambda i: (0, 2 * i)),
            pl.BlockSpec((D, _GRP), lambda i: (0, 2 * i + 1)),
            pl.BlockSpec((D, D), lambda i: (0, 0)),
            pl.BlockSpec((D,), lambda i: (0,)),
        ],
        out_specs=pl.BlockSpec((_GRP, 2 * D), lambda i: (i, 0)),
        out_shape=jax.ShapeDtypeStruct((_VP // 2, 2 * D), jnp.float32),
    )(tableT, tableT, W, b)


def _gather(N, D):
    per = N // _NW
    NCH = per // _CH
    assert per % _CH == 0 and NCH % 2 == 0
    mesh = plsc.VectorSubcoreMesh(core_axis_name="c", subcore_axis_name="s")

    HC = _CH // 2

    @functools.partial(
        pl.kernel,
        out_type=jax.ShapeDtypeStruct((N // _CH, HC, 2, D), jnp.float32),
        mesh=mesh,
        scratch_types=[
            pltpu.VMEM((NCH, _CH), jnp.int32),
            pltpu.VMEM((2, _CH, D), jnp.float32),
            pltpu.SemaphoreType.DMA,
            pltpu.SemaphoreType.DMA,
        ],
        compiler_params=pltpu.CompilerParams(use_tc_tiling_on_sc=False),
    )
    def gather_k(tok_hbm, t2_hbm, out_hbm, idx_v, rows_v, gsem, ssem):
        wid = lax.axis_index("s") * _NC + lax.axis_index("c")
        pltpu.sync_copy(tok_hbm.at[pl.ds(wid * NCH, NCH)], idx_v)

        def fire_g(c, p):
            pltpu.async_copy(t2_hbm.at[idx_v.at[c]], rows_v.at[p], gsem)

        def drain_g(c, p):
            pltpu.make_async_copy(
                t2_hbm.at[idx_v.at[c]], rows_v.at[p], gsem
            ).wait()

        def fire_s(c, p):
            for h in range(2):
                pltpu.async_copy(
                    rows_v.at[p, pl.ds(h * HC, HC)],
                    out_hbm.at[wid * NCH + c, :, h],
                    ssem,
                )

        def drain_s(c, p):
            for h in range(2):
                pltpu.make_async_copy(
                    rows_v.at[p, pl.ds(h * HC, HC)],
                    out_hbm.at[wid * NCH + c, :, h],
                    ssem,
                ).wait()

        fire_g(0, 0)
        drain_g(0, 0)
        fire_s(0, 0)
        fire_g(1, 1)

        def body(k, carry):
            c1 = 2 * k + 1
            drain_g(c1, 1)
            fire_s(c1, 1)
            drain_s(c1 - 1, 0)
            fire_g(c1 + 1, 0)
            c2 = 2 * k + 2
            drain_g(c2, 0)
            fire_s(c2, 0)
            drain_s(c2 - 1, 1)
            fire_g(c2 + 1, 1)
            return carry

        lax.fori_loop(0, NCH // 2 - 1, body, 0)

        drain_g(NCH - 1, 1)
        fire_s(NCH - 1, 1)
        drain_s(NCH - 2, 0)
        drain_s(NCH - 1, 1)

    return gather_k


def _untranspose(G128, L, D, Bsz):
    BH = Bsz // 2

    LB = 2

    def body(g_ref, i_ref, o_ref):
        g = g_ref[...]
        for j in range(LB):
            for h in range(2):
                o_ref[j, :, h * BH:(h + 1) * BH] = lax.dot_general(
                    i_ref[...],
                    g[j * BH:(j + 1) * BH, h * D:(h + 1) * D],
                    dimension_numbers=(((0,), (1,)), ((), ())),
                    preferred_element_type=jnp.float32,
                )

    return pl.pallas_call(
        body,
        grid=(L // LB,),
        in_specs=[
            pl.BlockSpec((LB * BH, 2 * D), lambda l: (l, 0)),
            pl.BlockSpec((D, D), lambda l: (0, 0)),
        ],
        out_specs=pl.BlockSpec((LB, D, Bsz), lambda l: (l, 0, 0)),
        out_shape=jax.ShapeDtypeStruct((L, D, Bsz), jnp.float32),
    )(G128, jnp.eye(D, dtype=jnp.float32))


def kernel(token_ids, table, W, b):
    Bsz, L = token_ids.shape
    V, D = table.shape
    N = Bsz * L
    BH = Bsz // 2
    assert N % (_NW * _CH) == 0 and V <= _VP

    tokT = token_ids.T
    k = tokT // (2 * _GRP)
    pos = tokT % (2 * _GRP)
    h = (pos >= _GRP).astype(jnp.int32)
    tokR = 2 * (k * _GRP + pos - _GRP * h) + h
    HC = _CH // 2
    tok2d = (
        tokR.reshape(L, 2, BH // HC, HC)
        .transpose(0, 2, 1, 3)
        .reshape(N // _CH, _CH)
    )

    t2p = _transform_pack(table.T, W, b)
    t2 = t2p.reshape(_VP, D)
    G3 = _gather(N, D)(tok2d, t2)
    G128 = G3.reshape(N // 2, 2 * D)
    outP = _untranspose(G128, L, D, Bsz)
    return jnp.transpose(outP, (2, 0, 1))


# speedup vs baseline: 2.3433x; 1.0500x over previous
import functools

import jax
import jax.numpy as jnp
from jax import lax
from jax.experimental import pallas as pl
from jax.experimental.pallas import tpu as pltpu
from jax.experimental.pallas import tpu_sc as plsc

_NC = 2
_NS = 16
_NW = _NC * _NS

_CH = 128
_GRP = 6400
_VP = 102400


def _transform_pack(tableT, W, b):
    D, V = tableT.shape
    grid = _VP // (2 * _GRP)

    def body(tl_ref, tr_ref, w_ref, b_ref, o_ref):
        bias = b_ref[...][None, :]
        for h, t_ref in enumerate((tl_ref, tr_ref)):
            o_ref[:, h * D:(h + 1) * D] = (
                lax.dot_general(
                    t_ref[...],
                    w_ref[...],
                    dimension_numbers=(((0,), (0,)), ((), ())),
                    preferred_element_type=jnp.float32,
                )
                + bias
            )

    return pl.pallas_call(
        body,
        grid=(grid,),
        in_specs=[
            pl.BlockSpec((D, _GRP), lambda i: (0, 2 * i)),
            pl.BlockSpec((D, _GRP), lambda i: (0, 2 * i + 1)),
            pl.BlockSpec((D, D), lambda i: (0, 0)),
            pl.BlockSpec((D,), lambda i: (0,)),
        ],
        out_specs=pl.BlockSpec((_GRP, 2 * D), lambda i: (i, 0)),
        out_shape=jax.ShapeDtypeStruct((_VP // 2, 2 * D), jnp.float32),
    )(tableT, tableT, W, b)


def _gather(N, D):
    per = N // _NW
    NCH = per // _CH
    assert per % _CH == 0 and NCH % 2 == 0
    mesh = plsc.VectorSubcoreMesh(core_axis_name="c", subcore_axis_name="s")

    HC = _CH // 2

    @functools.partial(
        pl.kernel,
        out_type=jax.ShapeDtypeStruct((N // _CH, HC, 2, D), jnp.float32),
        mesh=mesh,
        scratch_types=[
            pltpu.VMEM((NCH, _CH), jnp.int32),
            pltpu.VMEM((2, _CH, D), jnp.float32),
            pltpu.SemaphoreType.DMA,
            pltpu.SemaphoreType.DMA,
        ],
        compiler_params=pltpu.CompilerParams(use_tc_tiling_on_sc=False),
    )
    def gather_k(tok_hbm, t2_hbm, out_hbm, idx_v, rows_v, gsem, ssem):
        wid = lax.axis_index("s") * _NC + lax.axis_index("c")
        pltpu.sync_copy(tok_hbm.at[pl.ds(wid * NCH, NCH)], idx_v)

        def fire_g(c, p):
            pltpu.async_copy(t2_hbm.at[idx_v.at[c]], rows_v.at[p], gsem)

        def drain_g(c, p):
            pltpu.make_async_copy(
                t2_hbm.at[idx_v.at[c]], rows_v.at[p], gsem
            ).wait()

        def fire_s(c, p):
            for h in range(2):
                pltpu.async_copy(
                    rows_v.at[p, pl.ds(h * HC, HC)],
                    out_hbm.at[wid * NCH + c, :, h],
                    ssem,
                )

        def drain_s(c, p):
            for h in range(2):
                pltpu.make_async_copy(
                    rows_v.at[p, pl.ds(h * HC, HC)],
                    out_hbm.at[wid * NCH + c, :, h],
                    ssem,
                ).wait()

        fire_g(0, 0)
        drain_g(0, 0)
        fire_s(0, 0)
        fire_g(1, 1)

        def body(k, carry):
            c1 = 2 * k + 1
            drain_g(c1, 1)
            fire_s(c1, 1)
            drain_s(c1 - 1, 0)
            fire_g(c1 + 1, 0)
            c2 = 2 * k + 2
            drain_g(c2, 0)
            fire_s(c2, 0)
            drain_s(c2 - 1, 1)
            fire_g(c2 + 1, 1)
            return carry

        lax.fori_loop(0, NCH // 2 - 1, body, 0)

        drain_g(NCH - 1, 1)
        fire_s(NCH - 1, 1)
        drain_s(NCH - 2, 0)
        drain_s(NCH - 1, 1)

    return gather_k


def _untranspose(G128, L, D, Bsz):
    BH = Bsz // 2

    LB = 5

    def body(g_ref, i_ref, o_ref):
        g = g_ref[...]
        for j in range(LB):
            for h in range(2):
                o_ref[j, :, h * BH:(h + 1) * BH] = lax.dot_general(
                    i_ref[...],
                    g[j * BH:(j + 1) * BH, h * D:(h + 1) * D],
                    dimension_numbers=(((0,), (1,)), ((), ())),
                    preferred_element_type=jnp.float32,
                )

    return pl.pallas_call(
        body,
        grid=(L // LB,),
        in_specs=[
            pl.BlockSpec((LB * BH, 2 * D), lambda l: (l, 0)),
            pl.BlockSpec((D, D), lambda l: (0, 0)),
        ],
        out_specs=pl.BlockSpec((LB, D, Bsz), lambda l: (l, 0, 0)),
        out_shape=jax.ShapeDtypeStruct((L, D, Bsz), jnp.float32),
    )(G128, jnp.eye(D, dtype=jnp.float32))


def kernel(token_ids, table, W, b):
    Bsz, L = token_ids.shape
    V, D = table.shape
    N = Bsz * L
    BH = Bsz // 2
    assert N % (_NW * _CH) == 0 and V <= _VP

    tokT = token_ids.T
    k = tokT // (2 * _GRP)
    pos = tokT % (2 * _GRP)
    h = (pos >= _GRP).astype(jnp.int32)
    tokR = 2 * (k * _GRP + pos - _GRP * h) + h
    HC = _CH // 2
    tok2d = (
        tokR.reshape(L, 2, BH // HC, HC)
        .transpose(0, 2, 1, 3)
        .reshape(N // _CH, _CH)
    )

    t2p = _transform_pack(table.T, W, b)
    t2 = t2p.reshape(_VP, D)
    G3 = _gather(N, D)(tok2d, t2)
    G128 = G3.reshape(N // 2, 2 * D)
    outP = _untranspose(G128, L, D, Bsz)
    return jnp.transpose(outP, (2, 0, 1))


# speedup vs baseline: 2.3519x; 1.0037x over previous
import functools

import jax
import jax.numpy as jnp
from jax import lax
from jax.experimental import pallas as pl
from jax.experimental.pallas import tpu as pltpu
from jax.experimental.pallas import tpu_sc as plsc

_NC = 2
_NS = 16
_NW = _NC * _NS

_CH = 128
_GRP = 6400
_VP = 102400


def _transform_pack(tableT, W, b):
    D, V = tableT.shape
    grid = _VP // (2 * _GRP)

    def body(tl_ref, tr_ref, w_ref, b_ref, o_ref):
        bias = b_ref[...][None, :]
        for h, t_ref in enumerate((tl_ref, tr_ref)):
            o_ref[:, h * D:(h + 1) * D] = (
                lax.dot_general(
                    t_ref[...],
                    w_ref[...],
                    dimension_numbers=(((0,), (0,)), ((), ())),
                    preferred_element_type=jnp.float32,
                )
                + bias
            )

    return pl.pallas_call(
        body,
        grid=(grid,),
        in_specs=[
            pl.BlockSpec((D, _GRP), lambda i: (0, 2 * i)),
            pl.BlockSpec((D, _GRP), lambda i: (0, 2 * i + 1)),
            pl.BlockSpec((D, D), lambda i: (0, 0)),
            pl.BlockSpec((D,), lambda i: (0,)),
        ],
        out_specs=pl.BlockSpec((_GRP, 2 * D), lambda i: (i, 0)),
        out_shape=jax.ShapeDtypeStruct((_VP // 2, 2 * D), jnp.float32),
    )(tableT, tableT, W, b)


def _gather(N, D):
    per = N // _NW
    NCH = per // _CH
    assert per % _CH == 0 and NCH % 2 == 0
    mesh = plsc.VectorSubcoreMesh(core_axis_name="c", subcore_axis_name="s")

    HC = _CH // 2

    @functools.partial(
        pl.kernel,
        out_type=jax.ShapeDtypeStruct((N // _CH, HC, 2, D), jnp.float32),
        mesh=mesh,
        scratch_types=[
            pltpu.VMEM((NCH, _CH), jnp.int32),
            pltpu.VMEM((2, _CH, D), jnp.float32),
            pltpu.SemaphoreType.DMA,
            pltpu.SemaphoreType.DMA,
        ],
        compiler_params=pltpu.CompilerParams(use_tc_tiling_on_sc=False),
    )
    def gather_k(tok_hbm, t2_hbm, out_hbm, idx_v, rows_v, gsem, ssem):
        wid = lax.axis_index("s") * _NC + lax.axis_index("c")
        pltpu.sync_copy(tok_hbm.at[pl.ds(wid * NCH, NCH)], idx_v)

        def fire_g(c, p):
            pltpu.async_copy(t2_hbm.at[idx_v.at[c]], rows_v.at[p], gsem)

        def drain_g(c, p):
            pltpu.make_async_copy(
                t2_hbm.at[idx_v.at[c]], rows_v.at[p], gsem
            ).wait()

        def fire_s(c, p):
            for h in range(2):
                pltpu.async_copy(
                    rows_v.at[p, pl.ds(h * HC, HC)],
                    out_hbm.at[wid * NCH + c, :, h],
                    ssem,
                )

        def drain_s(c, p):
            for h in range(2):
                pltpu.make_async_copy(
                    rows_v.at[p, pl.ds(h * HC, HC)],
                    out_hbm.at[wid * NCH + c, :, h],
                    ssem,
                ).wait()

        fire_g(0, 0)
        drain_g(0, 0)
        fire_s(0, 0)
        fire_g(1, 1)

        def body(k, carry):
            c1 = 2 * k + 1
            drain_g(c1, 1)
            fire_s(c1, 1)
            drain_s(c1 - 1, 0)
            fire_g(c1 + 1, 0)
            c2 = 2 * k + 2
            drain_g(c2, 0)
            fire_s(c2, 0)
            drain_s(c2 - 1, 1)
            fire_g(c2 + 1, 1)
            return carry

        lax.fori_loop(0, NCH // 2 - 1, body, 0)

        drain_g(NCH - 1, 1)
        fire_s(NCH - 1, 1)
        drain_s(NCH - 2, 0)
        drain_s(NCH - 1, 1)

    return gather_k


def _untranspose(G128, L, D, Bsz):
    BH = Bsz // 2

    LB = 10

    def body(g_ref, i_ref, o_ref):
        g = g_ref[...]
        for j in range(LB):
            for h in range(2):
                o_ref[j, :, h * BH:(h + 1) * BH] = lax.dot_general(
                    i_ref[...],
                    g[j * BH:(j + 1) * BH, h * D:(h + 1) * D],
                    dimension_numbers=(((0,), (1,)), ((), ())),
                    preferred_element_type=jnp.float32,
                )

    return pl.pallas_call(
        body,
        grid=(L // LB,),
        in_specs=[
            pl.BlockSpec((LB * BH, 2 * D), lambda l: (l, 0)),
            pl.BlockSpec((D, D), lambda l: (0, 0)),
        ],
        out_specs=pl.BlockSpec((LB, D, Bsz), lambda l: (l, 0, 0)),
        out_shape=jax.ShapeDtypeStruct((L, D, Bsz), jnp.float32),
    )(G128, jnp.eye(D, dtype=jnp.float32))


def kernel(token_ids, table, W, b):
    Bsz, L = token_ids.shape
    V, D = table.shape
    N = Bsz * L
    BH = Bsz // 2
    assert N % (_NW * _CH) == 0 and V <= _VP

    tokT = token_ids.T
    k = tokT // (2 * _GRP)
    pos = tokT % (2 * _GRP)
    h = (pos >= _GRP).astype(jnp.int32)
    tokR = 2 * (k * _GRP + pos - _GRP * h) + h
    HC = _CH // 2
    tok2d = (
        tokR.reshape(L, 2, BH // HC, HC)
        .transpose(0, 2, 1, 3)
        .reshape(N // _CH, _CH)
    )

    t2p = _transform_pack(table.T, W, b)
    t2 = t2p.reshape(_VP, D)
    G3 = _gather(N, D)(tok2d, t2)
    G128 = G3.reshape(N // 2, 2 * D)
    outP = _untranspose(G128, L, D, Bsz)
    return jnp.transpose(outP, (2, 0, 1))
